# Initial kernel scaffold; baseline (speedup 1.0000x reference)
import jax
import jax.numpy as jnp
from jax.experimental import pallas as pl


def kernel(x, edge_index, batch, params):
    raise NotImplementedError("write your pallas kernel here")


# speedup vs baseline: 6.9640x; 6.9640x over previous
import functools

import jax
import jax.numpy as jnp
from jax import lax
from jax.experimental import pallas as pl
from jax.experimental.pallas import tpu as pltpu
from jax.experimental.pallas import tpu_sc as plsc

_N = 10000
_E = 320000
_D = 128
_G = 64
_NC = 2
_NS = 16
_NW = _NC * _NS
_EPW = _E // _NW
_K = 125
_CHUNKS = _EPW // _K
_NPAD = 10240
_RPT = _NPAD // _NS
_CW = 16

_R = 1000
_NB = _N // _R

@functools.lru_cache(maxsize=None)
def _sc_kernels():
    mesh = plsc.VectorSubcoreMesh(core_axis_name="c", subcore_axis_name="s",
                                  num_cores=_NC, num_subcores=_NS)

    @functools.partial(
        pl.kernel,
        out_type=jax.ShapeDtypeStruct((_NC, _NPAD, _D), jnp.float32),
        mesh=mesh,
        scratch_types=[
            pltpu.VMEM((_CHUNKS, _K), jnp.int32),
            pltpu.VMEM((_CHUNKS, _K), jnp.int32),
            pltpu.VMEM((_K, _D), jnp.float32),
            pltpu.VMEM_SHARED((_NPAD, _D), jnp.float32),
            pltpu.SemaphoreType.DMA,
        ],
    )
    def sc_segment_sum(h_hbm, src_hbm, dst_hbm, zeros_hbm, out_hbm,
                       src_v, dst_v, rows_v, acc_sh, sem):
        cid = lax.axis_index("c")
        sid = lax.axis_index("s")
        wid = cid * _NS + sid
        pltpu.sync_copy(zeros_hbm, acc_sh.at[pl.ds(sid * _RPT, _RPT)])
        base = wid * _CHUNKS
        pltpu.sync_copy(src_hbm.at[pl.ds(base, _CHUNKS)], src_v)
        pltpu.sync_copy(dst_hbm.at[pl.ds(base, _CHUNKS)], dst_v)
        plsc.subcore_barrier()

        def body(j, carry):
            pltpu.async_copy(h_hbm.at[src_v.at[j]], rows_v, sem).wait()
            pltpu.sync_copy(rows_v, acc_sh.at[dst_v.at[j]], add=True)
            return carry

        lax.fori_loop(0, _CHUNKS, body, 0)
        plsc.subcore_barrier()
        pltpu.sync_copy(acc_sh.at[pl.ds(sid * _RPT, _RPT)],
                        out_hbm.at[cid, pl.ds(sid * _RPT, _RPT)])

    @functools.partial(
        pl.kernel,
        out_type=jax.ShapeDtypeStruct((_NC, _NPAD, _CW), jnp.float32),
        mesh=mesh,
        scratch_types=[
            pltpu.VMEM((_CHUNKS, _K), jnp.int32),
            pltpu.VMEM((_K, _CW), jnp.float32),
            pltpu.VMEM_SHARED((_NPAD, _CW), jnp.float32),
        ],
    )
    def sc_degree(dst_hbm, ones_hbm, zeros_hbm, out_hbm, dst_v, ones_v, acc_sh):
        cid = lax.axis_index("c")
        sid = lax.axis_index("s")
        wid = cid * _NS + sid
        pltpu.sync_copy(zeros_hbm, acc_sh.at[pl.ds(sid * _RPT, _RPT)])
        pltpu.sync_copy(ones_hbm, ones_v)
        pltpu.sync_copy(dst_hbm.at[pl.ds(wid * _CHUNKS, _CHUNKS)], dst_v)
        plsc.subcore_barrier()

        def body(j, carry):
            pltpu.sync_copy(ones_v, acc_sh.at[dst_v.at[j]], add=True)
            return carry

        lax.fori_loop(0, _CHUNKS, body, 0)
        plsc.subcore_barrier()
        pltpu.sync_copy(acc_sh.at[pl.ds(sid * _RPT, _RPT)],
                        out_hbm.at[cid, pl.ds(sid * _RPT, _RPT)])

    return sc_segment_sum, sc_degree


def _conv_body(s0, s1, c0, c1, h, wlT, bl, wrT, t_ref, st_ref):
    i = pl.program_id(0)
    s = s0[...] + s1[...]
    cnt = c0[:, 0:1] + c1[:, 0:1]
    inv = 1.0 / jnp.maximum(cnt, 1.0)
    t = (inv * jnp.dot(s, wlT[...], preferred_element_type=jnp.float32)
         + bl[...]
         + jnp.dot(h[...], wrT[...], preferred_element_type=jnp.float32))
    t_ref[...] = t

    @pl.when(i == 0)
    def _():
        st_ref[...] = jnp.zeros((8, _D), jnp.float32)

    upd = jnp.concatenate(
        [jnp.sum(t, axis=0)[None, :], jnp.sum(t * t, axis=0)[None, :],
         jnp.zeros((6, _D), jnp.float32)], axis=0)
    st_ref[...] += upd


def _tc_conv(s0, s1, c0, c1, h, wlT, bl, wrT):
    return pl.pallas_call(
        _conv_body,
        grid=(_NB,),
        in_specs=[
            pl.BlockSpec((_R, _D), lambda i: (i, 0)),
            pl.BlockSpec((_R, _D), lambda i: (i, 0)),
            pl.BlockSpec((_R, _D), lambda i: (i, 0)),
            pl.BlockSpec((_R, _D), lambda i: (i, 0)),
            pl.BlockSpec((_R, _D), lambda i: (i, 0)),
            pl.BlockSpec((_D, _D), lambda i: (0, 0)),
            pl.BlockSpec((1, _D), lambda i: (0, 0)),
            pl.BlockSpec((_D, _D), lambda i: (0, 0)),
        ],
        out_specs=[
            pl.BlockSpec((_R, _D), lambda i: (i, 0)),
            pl.BlockSpec((8, _D), lambda i: (0, 0)),
        ],
        out_shape=[
            jax.ShapeDtypeStruct((_N, _D), jnp.float32),
            jax.ShapeDtypeStruct((8, _D), jnp.float32),
        ],
    )(s0, s1, c0, c1, h, wlT, bl, wrT)


def _bn_body(t, st, gamma, beta, o_ref):
    stt = st[...]
    mu = stt[0:1, :] * (1.0 / _N)
    var = stt[1:2, :] * (1.0 / _N) - mu * mu
    scale = gamma[...] / jnp.sqrt(var + 1e-5)
    shift = beta[...] - mu * scale
    o_ref[...] = jnp.maximum(t[...] * scale + shift, 0.0)


def _tc_bn_relu(t, st, gamma, beta):
    return pl.pallas_call(
        _bn_body,
        grid=(_NB,),
        in_specs=[
            pl.BlockSpec((_R, _D), lambda i: (i, 0)),
            pl.BlockSpec((8, _D), lambda i: (0, 0)),
            pl.BlockSpec((1, _D), lambda i: (0, 0)),
            pl.BlockSpec((1, _D), lambda i: (0, 0)),
        ],
        out_specs=pl.BlockSpec((_R, _D), lambda i: (i, 0)),
        out_shape=jax.ShapeDtypeStruct((_N, _D), jnp.float32),
    )(t, st, gamma, beta)


def _pool_body(s0, s1, c0, c1, h, wlT, bl, wrT, batchb, o_ref, acc_s, acc_c):
    i = pl.program_id(0)
    s = s0[...] + s1[...]
    cnt = c0[:, 0:1] + c1[:, 0:1]
    inv = 1.0 / jnp.maximum(cnt, 1.0)
    t = (inv * jnp.dot(s, wlT[...], preferred_element_type=jnp.float32)
         + bl[...]
         + jnp.dot(h[...], wrT[...], preferred_element_type=jnp.float32))
    b = batchb[...].reshape(_R)
    mask_t = (lax.broadcasted_iota(jnp.int32, (_G, _R), 0)
              == b[None, :]).astype(jnp.float32)

    @pl.when(i == 0)
    def _():
        acc_s[...] = jnp.zeros((_G, _D), jnp.float32)
        acc_c[...] = jnp.zeros((_G, _D), jnp.float32)

    acc_s[...] += jnp.dot(mask_t, t, preferred_element_type=jnp.float32)
    acc_c[...] += jnp.dot(mask_t, jnp.ones((_R, _D), jnp.float32),
                          preferred_element_type=jnp.float32)

    @pl.when(i == _NB - 1)
    def _():
        o_ref[...] = acc_s[...] / jnp.maximum(acc_c[...], 1.0)


def _tc_conv_pool(s0, s1, c0, c1, h, wlT, bl, wrT, batch3):
    return pl.pallas_call(
        _pool_body,
        grid=(_NB,),
        in_specs=[
            pl.BlockSpec((_R, _D), lambda i: (i, 0)),
            pl.BlockSpec((_R, _D), lambda i: (i, 0)),
            pl.BlockSpec((_R, _D), lambda i: (i, 0)),
            pl.BlockSpec((_R, _D), lambda i: (i, 0)),
            pl.BlockSpec((_R, _D), lambda i: (i, 0)),
            pl.BlockSpec((_D, _D), lambda i: (0, 0)),
            pl.BlockSpec((1, _D), lambda i: (0, 0)),
            pl.BlockSpec((_D, _D), lambda i: (0, 0)),
            pl.BlockSpec((1, 1, _R), lambda i: (i, 0, 0)),
        ],
        out_specs=pl.BlockSpec((_G, _D), lambda i: (0, 0)),
        out_shape=jax.ShapeDtypeStruct((_G, _D), jnp.float32),
        scratch_shapes=[
            pltpu.VMEM((_G, _D), jnp.float32),
            pltpu.VMEM((_G, _D), jnp.float32),
        ],
    )(s0, s1, c0, c1, h, wlT, bl, wrT, batch3)


def kernel(x, edge_index, batch, params):
    src = edge_index[0].reshape(_E // _K, _K)
    dst = edge_index[1].reshape(_E // _K, _K)
    batch3 = batch.reshape(_NB, 1, _R)
    zeros_rows = jnp.zeros((_RPT, _D), jnp.float32)
    ones_table = jnp.ones((_N, _D), jnp.float32)

    sc_segment_sum, _ = _sc_kernels()
    cnt2 = sc_segment_sum(ones_table, src, dst, zeros_rows)
    c0, c1 = cnt2[0], cnt2[1]

    h = x
    for li, layer in enumerate(params):
        wlT = layer['Wl'].T
        wrT = layer['Wr'].T
        bl = layer['bl'].reshape(1, _D)
        s2 = sc_segment_sum(h, src, dst, zeros_rows)
        s0, s1 = s2[0], s2[1]
        if li < len(params) - 1:
            t, st = _tc_conv(s0, s1, c0, c1, h, wlT, bl, wrT)
            h = _tc_bn_relu(t, st, layer['gamma'].reshape(1, _D),
                            layer['beta'].reshape(1, _D))
        else:
            h = _tc_conv_pool(s0, s1, c0, c1, h, wlT, bl, wrT, batch3)
    return h


# speedup vs baseline: 9.0831x; 1.3043x over previous
import functools

import jax
import jax.numpy as jnp
from jax import lax
from jax.experimental import pallas as pl
from jax.experimental.pallas import tpu as pltpu
from jax.experimental.pallas import tpu_sc as plsc

_N = 10000
_E = 320000
_D = 128
_G = 64
_NC = 2
_NS = 16
_NW = _NC * _NS
_EPW = _E // _NW
_K = 125
_CHUNKS = _EPW // _K
_NPAD = 10240
_RPT = _NPAD // _NS
_CW = 16
_GC = 16

_R = 1000
_NB = _N // _R

@functools.lru_cache(maxsize=None)
def _sc_kernels():
    mesh = plsc.VectorSubcoreMesh(core_axis_name="c", subcore_axis_name="s",
                                  num_cores=_NC, num_subcores=_NS)

    @functools.partial(
        pl.kernel,
        out_type=jax.ShapeDtypeStruct((_NC, _NPAD, _D), jnp.float32),
        mesh=mesh,
        scratch_types=[
            pltpu.VMEM((_GC, _K), jnp.int32),
            pltpu.VMEM((_GC, _K), jnp.int32),
            pltpu.VMEM((2, _K, _D), jnp.float32),
            pltpu.VMEM_SHARED((_NPAD, _D), jnp.float32),
            pltpu.SemaphoreType.DMA,
            pltpu.SemaphoreType.DMA,
        ],
    )
    def sc_segment_sum(h_hbm, src_hbm, dst_hbm, zeros_hbm, out_hbm,
                       src_v, dst_v, rows_v, acc_sh, sem0, sem1):
        cid = lax.axis_index("c")
        sid = lax.axis_index("s")
        wid = cid * _NS + sid
        pltpu.sync_copy(zeros_hbm, acc_sh.at[pl.ds(sid * _RPT, _RPT)])
        base = wid * _CHUNKS
        plsc.subcore_barrier()

        def group(g, carry):
            gbase = base + g * _GC
            pltpu.sync_copy(src_hbm.at[pl.ds(gbase, _GC)], src_v)
            pltpu.sync_copy(dst_hbm.at[pl.ds(gbase, _GC)], dst_v)
            pltpu.async_copy(h_hbm.at[src_v.at[0]], rows_v.at[0], sem0)

            def body(m, carry2):
                j0 = m * 2
                j1 = j0 + 1
                pltpu.make_async_copy(h_hbm.at[src_v.at[j0]], rows_v.at[0],
                                      sem0).wait()
                pltpu.async_copy(h_hbm.at[src_v.at[j1]], rows_v.at[1], sem1)
                pltpu.sync_copy(rows_v.at[0], acc_sh.at[dst_v.at[j0]],
                                add=True)
                pltpu.make_async_copy(h_hbm.at[src_v.at[j1]], rows_v.at[1],
                                      sem1).wait()

                @pl.when(m < _GC // 2 - 1)
                def _():
                    pltpu.async_copy(h_hbm.at[src_v.at[j0 + 2]], rows_v.at[0],
                                     sem0)

                pltpu.sync_copy(rows_v.at[1], acc_sh.at[dst_v.at[j1]],
                                add=True)
                return carry2

            lax.fori_loop(0, _GC // 2, body, carry)
            return carry

        lax.fori_loop(0, _CHUNKS // _GC, group, 0)
        plsc.subcore_barrier()
        pltpu.sync_copy(acc_sh.at[pl.ds(sid * _RPT, _RPT)],
                        out_hbm.at[cid, pl.ds(sid * _RPT, _RPT)])

    @functools.partial(
        pl.kernel,
        out_type=jax.ShapeDtypeStruct((_NC, _NPAD, _D), jnp.float32),
        mesh=mesh,
        scratch_types=[
            pltpu.VMEM((_GC, _K), jnp.int32),
            pltpu.VMEM((_K, _D), jnp.float32),
            pltpu.VMEM_SHARED((_NPAD, _D), jnp.float32),
        ],
    )
    def sc_degree(dst_hbm, ones_hbm, zeros_hbm, out_hbm,
                  dst_v, ones_v, acc_sh):
        cid = lax.axis_index("c")
        sid = lax.axis_index("s")
        wid = cid * _NS + sid
        pltpu.sync_copy(zeros_hbm, acc_sh.at[pl.ds(sid * _RPT, _RPT)])
        pltpu.sync_copy(ones_hbm, ones_v)
        base = wid * _CHUNKS
        plsc.subcore_barrier()

        def group(g, carry):
            pltpu.sync_copy(dst_hbm.at[pl.ds(base + g * _GC, _GC)], dst_v)

            def body(j, carry2):
                pltpu.sync_copy(ones_v, acc_sh.at[dst_v.at[j]], add=True)
                return carry2

            lax.fori_loop(0, _GC, body, carry)
            return carry

        lax.fori_loop(0, _CHUNKS // _GC, group, 0)
        plsc.subcore_barrier()
        pltpu.sync_copy(acc_sh.at[pl.ds(sid * _RPT, _RPT)],
                        out_hbm.at[cid, pl.ds(sid * _RPT, _RPT)])

    return sc_segment_sum, sc_degree


def _conv_body(s0, s1, c0, c1, h, wlT, bl, wrT, t_ref, st_ref):
    i = pl.program_id(0)
    s = s0[...] + s1[...]
    cnt = c0[...] + c1[...]
    inv = 1.0 / jnp.maximum(cnt, 1.0)
    t = (inv * jnp.dot(s, wlT[...], preferred_element_type=jnp.float32)
         + bl[...]
         + jnp.dot(h[...], wrT[...], preferred_element_type=jnp.float32))
    t_ref[...] = t

    @pl.when(i == 0)
    def _():
        st_ref[...] = jnp.zeros((8, _D), jnp.float32)

    upd = jnp.concatenate(
        [jnp.sum(t, axis=0)[None, :], jnp.sum(t * t, axis=0)[None, :],
         jnp.zeros((6, _D), jnp.float32)], axis=0)
    st_ref[...] += upd


def _tc_conv(s0, s1, c0, c1, h, wlT, bl, wrT):
    return pl.pallas_call(
        _conv_body,
        grid=(_NB,),
        in_specs=[
            pl.BlockSpec((_R, _D), lambda i: (i, 0)),
            pl.BlockSpec((_R, _D), lambda i: (i, 0)),
            pl.BlockSpec((_R, 1), lambda i: (i, 0)),
            pl.BlockSpec((_R, 1), lambda i: (i, 0)),
            pl.BlockSpec((_R, _D), lambda i: (i, 0)),
            pl.BlockSpec((_D, _D), lambda i: (0, 0)),
            pl.BlockSpec((1, _D), lambda i: (0, 0)),
            pl.BlockSpec((_D, _D), lambda i: (0, 0)),
        ],
        out_specs=[
            pl.BlockSpec((_R, _D), lambda i: (i, 0)),
            pl.BlockSpec((8, _D), lambda i: (0, 0)),
        ],
        out_shape=[
            jax.ShapeDtypeStruct((_N, _D), jnp.float32),
            jax.ShapeDtypeStruct((8, _D), jnp.float32),
        ],
    )(s0, s1, c0, c1, h, wlT, bl, wrT)


def _bn_body(t, st, gamma, beta, o_ref):
    stt = st[...]
    mu = stt[0:1, :] * (1.0 / _N)
    var = stt[1:2, :] * (1.0 / _N) - mu * mu
    scale = gamma[...] / jnp.sqrt(var + 1e-5)
    shift = beta[...] - mu * scale
    o_ref[...] = jnp.maximum(t[...] * scale + shift, 0.0)


def _tc_bn_relu(t, st, gamma, beta):
    return pl.pallas_call(
        _bn_body,
        grid=(_NB,),
        in_specs=[
            pl.BlockSpec((_R, _D), lambda i: (i, 0)),
            pl.BlockSpec((8, _D), lambda i: (0, 0)),
            pl.BlockSpec((1, _D), lambda i: (0, 0)),
            pl.BlockSpec((1, _D), lambda i: (0, 0)),
        ],
        out_specs=pl.BlockSpec((_R, _D), lambda i: (i, 0)),
        out_shape=jax.ShapeDtypeStruct((_N, _D), jnp.float32),
    )(t, st, gamma, beta)


def _pool_body(s0, s1, c0, c1, h, wlT, bl, wrT, batchb, o_ref, acc_s, acc_c):
    i = pl.program_id(0)
    s = s0[...] + s1[...]
    cnt = c0[...] + c1[...]
    inv = 1.0 / jnp.maximum(cnt, 1.0)
    t = (inv * jnp.dot(s, wlT[...], preferred_element_type=jnp.float32)
         + bl[...]
         + jnp.dot(h[...], wrT[...], preferred_element_type=jnp.float32))
    b = batchb[...].reshape(_R)
    mask_t = (lax.broadcasted_iota(jnp.int32, (_G, _R), 0)
              == b[None, :]).astype(jnp.float32)

    @pl.when(i == 0)
    def _():
        acc_s[...] = jnp.zeros((_G, _D), jnp.float32)
        acc_c[...] = jnp.zeros((_G, _D), jnp.float32)

    acc_s[...] += jnp.dot(mask_t, t, preferred_element_type=jnp.float32)
    acc_c[...] += jnp.dot(mask_t, jnp.ones((_R, _D), jnp.float32),
                          preferred_element_type=jnp.float32)

    @pl.when(i == _NB - 1)
    def _():
        o_ref[...] = acc_s[...] / jnp.maximum(acc_c[...], 1.0)


def _tc_conv_pool(s0, s1, c0, c1, h, wlT, bl, wrT, batch3):
    return pl.pallas_call(
        _pool_body,
        grid=(_NB,),
        in_specs=[
            pl.BlockSpec((_R, _D), lambda i: (i, 0)),
            pl.BlockSpec((_R, _D), lambda i: (i, 0)),
            pl.BlockSpec((_R, 1), lambda i: (i, 0)),
            pl.BlockSpec((_R, 1), lambda i: (i, 0)),
            pl.BlockSpec((_R, _D), lambda i: (i, 0)),
            pl.BlockSpec((_D, _D), lambda i: (0, 0)),
            pl.BlockSpec((1, _D), lambda i: (0, 0)),
            pl.BlockSpec((_D, _D), lambda i: (0, 0)),
            pl.BlockSpec((1, 1, _R), lambda i: (i, 0, 0)),
        ],
        out_specs=pl.BlockSpec((_G, _D), lambda i: (0, 0)),
        out_shape=jax.ShapeDtypeStruct((_G, _D), jnp.float32),
        scratch_shapes=[
            pltpu.VMEM((_G, _D), jnp.float32),
            pltpu.VMEM((_G, _D), jnp.float32),
        ],
    )(s0, s1, c0, c1, h, wlT, bl, wrT, batch3)


def kernel(x, edge_index, batch, params):
    src = edge_index[0].reshape(_E // _K, _K)
    dst = edge_index[1].reshape(_E // _K, _K)
    batch3 = batch.reshape(_NB, 1, _R)
    zeros_rows = jnp.zeros((_RPT, _D), jnp.float32)
    ones_rows = jnp.ones((_K, _D), jnp.float32)

    sc_segment_sum, sc_degree = _sc_kernels()
    cnt2 = sc_degree(dst, ones_rows, zeros_rows)
    c0 = cnt2[0, :_N, :1]
    c1 = cnt2[1, :_N, :1]

    h = x
    for li, layer in enumerate(params):
        wlT = layer['Wl'].T
        wrT = layer['Wr'].T
        bl = layer['bl'].reshape(1, _D)
        s2 = sc_segment_sum(h, src, dst, zeros_rows)
        s0, s1 = s2[0], s2[1]
        if li < len(params) - 1:
            t, st = _tc_conv(s0, s1, c0, c1, h, wlT, bl, wrT)
            h = _tc_bn_relu(t, st, layer['gamma'].reshape(1, _D),
                            layer['beta'].reshape(1, _D))
        else:
            h = _tc_conv_pool(s0, s1, c0, c1, h, wlT, bl, wrT, batch3)
    return h
